# Initial kernel scaffold; baseline (speedup 1.0000x reference)
import jax
import jax.numpy as jnp
from jax.experimental import pallas as pl


def kernel(x, conv0_w, norm0_s, norm0_b, sb1_0, w1_0, s2_0, b2_0, w2_0, trans_s_0, trans_b_0, trans_w_0, sb1_1, w1_1, s2_1, b2_1, w2_1, trans_s_1, trans_b_1, trans_w_1, sb1_2, w1_2, s2_2, b2_2, w2_2, trans_s_2, trans_b_2, trans_w_2, sb1_3, w1_3, s2_3, b2_3, w2_3, norm5_s, norm5_b, lin_w, lin_b):
    raise NotImplementedError("write your pallas kernel here")


# speedup vs baseline: 1.1621x; 1.1621x over previous
import numpy as np

import jax
import jax.numpy as jnp
from jax import lax
from jax.experimental import pallas as pl
from jax.experimental.pallas import tpu as pltpu

GROWTH = 32
BOTTLENECK = 128
DB_CONFIG = (6, 12, 32, 32)
NUM_INIT_FEATURES = 64
NUM_CLASSES = 6
VMEM_LIMIT_BYTES = 48 * 1024 * 1024
TAPS = tuple((di, dj) for di in (-1, 0, 1) for dj in (-1, 0, 1))


def _cp(sem):
    return pltpu.CompilerParams(dimension_semantics=sem,
                                vmem_limit_bytes=VMEM_LIMIT_BYTES)


def _stem_body(col_ref, w_ref, s_ref, b_ref, o_ref):
    H = W = 64
    M = H * W
    y = jnp.dot(col_ref[...], w_ref[...], preferred_element_type=jnp.float32)
    y = jnp.maximum(y * s_ref[...] + b_ref[...], 0.0)
    r = lax.broadcasted_iota(jnp.int32, (M, NUM_INIT_FEATURES), 0)
    hh = r // W
    ww = r % W
    up = pltpu.roll(y, W, 0) * (hh >= 1).astype(jnp.float32)
    dn = pltpu.roll(y, M - W, 0) * (hh < H - 1).astype(jnp.float32)
    mv = jnp.maximum(y, jnp.maximum(up, dn))
    lf = pltpu.roll(mv, 1, 0) * (ww >= 1).astype(jnp.float32)
    rt = pltpu.roll(mv, M - 1, 0) * (ww < W - 1).astype(jnp.float32)
    mh = jnp.maximum(mv, jnp.maximum(lf, rt))
    sel = mh.reshape(H // 2, 2, W // 2, 2, NUM_INIT_FEATURES)[:, 0, :, 0, :]
    o_ref[...] = sel.reshape(M // 4, NUM_INIT_FEATURES)


def _stem_call(col_bf16, w, s, b, n_img):
    Mi = col_bf16.shape[0] // n_img
    K = col_bf16.shape[1]
    return pl.pallas_call(
        _stem_body,
        out_shape=jax.ShapeDtypeStruct((n_img * Mi // 4, NUM_INIT_FEATURES),
                                       jnp.float32),
        grid=(n_img,),
        in_specs=[pl.BlockSpec((Mi, K), lambda i: (i, 0)),
                  pl.BlockSpec((K, NUM_INIT_FEATURES), lambda i: (0, 0)),
                  pl.BlockSpec((1, NUM_INIT_FEATURES), lambda i: (0, 0)),
                  pl.BlockSpec((1, NUM_INIT_FEATURES), lambda i: (0, 0))],
        out_specs=pl.BlockSpec((Mi // 4, NUM_INIT_FEATURES), lambda i: (i, 0)),
        compiler_params=_cp(("arbitrary",)),
    )(col_bf16, w, s, b)


def _im2col7(x_nhwc):
    N, H, W, C = x_nhwc.shape
    xp = jnp.pad(x_nhwc, ((0, 0), (3, 3), (3, 3), (0, 0)))
    Ho = (H + 6 - 7) // 2 + 1
    cols = [xp[:, i:i + 2 * Ho:2, j:j + 2 * Ho:2, :]
            for i in range(7) for j in range(7)]
    return jnp.concatenate(cols, axis=-1).reshape(N * Ho * Ho, 49 * C)


def _tap_masks(H, W, M):
    r = lax.broadcasted_iota(jnp.int32, (M, GROWTH), 0)
    hh = (r // W) % H
    ww = r % W
    mh = {-1: (hh >= 1).astype(jnp.float32), 0: None,
          1: (hh < H - 1).astype(jnp.float32)}
    mw = {-1: (ww >= 1).astype(jnp.float32), 0: None,
          1: (ww < W - 1).astype(jnp.float32)}
    masks = {}
    for (di, dj) in TAPS:
        if (di, dj) == (0, 0):
            continue
        a, b = mh[di], mw[dj]
        masks[(di, dj)] = b if a is None else (a if b is None else a * b)
    return masks


def _dense_layers(feat_ref, sb1_ref, w1_ref, s2_ref, b2_ref, w2_ref,
                  base, L, H, W):
    M = feat_ref.shape[0]
    masks = _tap_masks(H, W, M)
    cins = [base + l * GROWTH for l in range(L)]
    offs = np.cumsum([0] + cins[:-1]).tolist()
    for l in range(L):
        cin, off = cins[l], offs[l]
        xin = feat_ref[:, :cin]
        h1 = jnp.maximum(xin * sb1_ref[0:1, off:off + cin]
                         + sb1_ref[1:2, off:off + cin], 0.0)
        y = jnp.dot(h1.astype(jnp.bfloat16), w1_ref[off:off + cin, :],
                    preferred_element_type=jnp.float32)
        z = jnp.maximum(y * s2_ref[l] + b2_ref[l], 0.0)
        zw = jnp.dot(z.astype(jnp.bfloat16), w2_ref[l],
                     preferred_element_type=jnp.float32)
        new = zw[:, 4 * GROWTH:5 * GROWTH]
        for ti, (di, dj) in enumerate(TAPS):
            if (di, dj) == (0, 0):
                continue
            sft = di * W + dj
            part = pltpu.roll(zw[:, ti * GROWTH:(ti + 1) * GROWTH],
                              (-sft) % M, 0)
            new = new + part * masks[(di, dj)]
        feat_ref[:, cin:cin + GROWTH] = new


def _avgpool_matmul(feat, ts, tb, tw, n_img, H, W, c_full):
    M = feat.shape[0]
    hf = jnp.maximum(feat * ts + tb, 0.0)
    a = hf + pltpu.roll(hf, M - 1, 0)
    s4 = a + pltpu.roll(a, M - W, 0)
    sel = s4.reshape(n_img * H // 2, 2, W // 2, 2, c_full)[:, 0, :, 0, :]
    pooled = sel.reshape(M // 4, c_full) * 0.25
    return jnp.dot(pooled.astype(jnp.bfloat16), tw,
                   preferred_element_type=jnp.float32)


def _make_block_call(base, L, H, W, n_img, has_trans):
    M = n_img * H * W
    c_full = base + L * GROWTH
    K_total = sum(base + l * GROWTH for l in range(L))

    if has_trans:
        def body(x_ref, sb1_ref, w1_ref, s2_ref, b2_ref, w2_ref,
                 ts_ref, tb_ref, tw_ref, o_ref, feat_ref):
            feat_ref[:, :base] = x_ref[...]
            _dense_layers(feat_ref, sb1_ref, w1_ref, s2_ref, b2_ref, w2_ref,
                          base, L, H, W)
            o_ref[...] = _avgpool_matmul(feat_ref[...], ts_ref[...],
                                         tb_ref[...], tw_ref[...],
                                         n_img, H, W, c_full)
    else:
        def body(x_ref, sb1_ref, w1_ref, s2_ref, b2_ref, w2_ref,
                 s5_ref, b5_ref, lw_ref, lb_ref, o_ref, feat_ref):
            feat_ref[:, :base] = x_ref[...]
            _dense_layers(feat_ref, sb1_ref, w1_ref, s2_ref, b2_ref, w2_ref,
                          base, L, H, W)
            hf = jnp.maximum(feat_ref[...] * s5_ref[...] + b5_ref[...], 0.0)
            pooled = hf.reshape(n_img, H * W, c_full).sum(axis=1) / (H * W)
            o_ref[...] = jnp.dot(pooled.astype(jnp.bfloat16), lw_ref[...],
                                 preferred_element_type=jnp.float32) + lb_ref[...]

    in_specs = [pl.BlockSpec((M, base), lambda i: (0, 0)),
                pl.BlockSpec((2, K_total), lambda i: (0, 0)),
                pl.BlockSpec((K_total, BOTTLENECK), lambda i: (0, 0)),
                pl.BlockSpec((L, 1, BOTTLENECK), lambda i: (0, 0, 0)),
                pl.BlockSpec((L, 1, BOTTLENECK), lambda i: (0, 0, 0)),
                pl.BlockSpec((L, BOTTLENECK, 9 * GROWTH), lambda i: (0, 0, 0))]
    if has_trans:
        c_half = c_full // 2
        in_specs += [pl.BlockSpec((1, c_full), lambda i: (0, 0)),
                     pl.BlockSpec((1, c_full), lambda i: (0, 0)),
                     pl.BlockSpec((c_full, c_half), lambda i: (0, 0))]
        out_shape = jax.ShapeDtypeStruct((M // 4, c_half), jnp.float32)
        out_specs = pl.BlockSpec((M // 4, c_half), lambda i: (0, 0))
    else:
        in_specs += [pl.BlockSpec((1, c_full), lambda i: (0, 0)),
                     pl.BlockSpec((1, c_full), lambda i: (0, 0)),
                     pl.BlockSpec((c_full, NUM_CLASSES), lambda i: (0, 0)),
                     pl.BlockSpec((1, NUM_CLASSES), lambda i: (0, 0))]
        out_shape = jax.ShapeDtypeStruct((n_img, NUM_CLASSES), jnp.float32)
        out_specs = pl.BlockSpec((n_img, NUM_CLASSES), lambda i: (0, 0))

    def call(feat, *params):
        return pl.pallas_call(
            body,
            out_shape=out_shape,
            grid=(1,),
            in_specs=in_specs,
            out_specs=out_specs,
            scratch_shapes=[pltpu.VMEM((M, c_full), jnp.float32)],
            compiler_params=_cp(("arbitrary",)),
        )(feat, *params)

    return call


def _w2cat(w2):
    L = w2.shape[0]
    return jnp.transpose(w2, (0, 2, 1, 3)).reshape(L, BOTTLENECK, 9 * GROWTH)


def kernel(x, conv0_w, norm0_s, norm0_b,
           sb1_0, w1_0, s2_0, b2_0, w2_0, trans_s_0, trans_b_0, trans_w_0,
           sb1_1, w1_1, s2_1, b2_1, w2_1, trans_s_1, trans_b_1, trans_w_1,
           sb1_2, w1_2, s2_2, b2_2, w2_2, trans_s_2, trans_b_2, trans_w_2,
           sb1_3, w1_3, s2_3, b2_3, w2_3,
           norm5_s, norm5_b, lin_w, lin_b):
    n_img = x.shape[0]
    xh = jnp.transpose(x, (0, 2, 3, 1)).astype(jnp.float32)
    col = _im2col7(xh).astype(jnp.bfloat16)
    feat = _stem_call(col, conv0_w, norm0_s, norm0_b, n_img)

    H = W = 32
    base = NUM_INIT_FEATURES
    block_params = [
        (sb1_0, w1_0, s2_0, b2_0, w2_0, trans_s_0, trans_b_0, trans_w_0),
        (sb1_1, w1_1, s2_1, b2_1, w2_1, trans_s_1, trans_b_1, trans_w_1),
        (sb1_2, w1_2, s2_2, b2_2, w2_2, trans_s_2, trans_b_2, trans_w_2),
        (sb1_3, w1_3, s2_3, b2_3, w2_3, norm5_s, norm5_b, lin_w, lin_b),
    ]
    for bi, L in enumerate(DB_CONFIG):
        p = block_params[bi]
        call = _make_block_call(base, L, H, W, n_img, has_trans=bi < 3)
        feat = call(feat, p[0], p[1], p[2], p[3], _w2cat(p[4]), *p[5:])
        if bi < 3:
            base = (base + L * GROWTH) // 2
            H //= 2
            W //= 2
    return feat


# speedup vs baseline: 1.1674x; 1.0046x over previous
import numpy as np

import jax
import jax.numpy as jnp
from jax import lax
from jax.experimental import pallas as pl
from jax.experimental.pallas import tpu as pltpu

GROWTH = 32
BOTTLENECK = 128
DB_CONFIG = (6, 12, 32, 32)
NUM_INIT_FEATURES = 64
NUM_CLASSES = 6
VMEM_LIMIT_BYTES = 64 * 1024 * 1024
TAPS = tuple((di, dj) for di in (-1, 0, 1) for dj in (-1, 0, 1))


def _im2col7(x_nhwc):
    N, H, W, C = x_nhwc.shape
    xp = jnp.pad(x_nhwc, ((0, 0), (3, 3), (3, 3), (0, 0)))
    Ho = (H + 6 - 7) // 2 + 1
    cols = [xp[:, i:i + 2 * Ho:2, j:j + 2 * Ho:2, :]
            for i in range(7) for j in range(7)]
    return jnp.concatenate(cols, axis=-1).reshape(N * Ho * Ho, 49 * C)


def _tap_masks(H, W, M):
    r = lax.broadcasted_iota(jnp.int32, (M, GROWTH), 0)
    hh = (r // W) % H
    ww = r % W
    mh = {-1: (hh >= 1).astype(jnp.float32), 0: None,
          1: (hh < H - 1).astype(jnp.float32)}
    mw = {-1: (ww >= 1).astype(jnp.float32), 0: None,
          1: (ww < W - 1).astype(jnp.float32)}
    masks = {}
    for (di, dj) in TAPS:
        if (di, dj) == (0, 0):
            continue
        a, b = mh[di], mw[dj]
        masks[(di, dj)] = b if a is None else (a if b is None else a * b)
    return masks


def _stem_compute(col_ref, c0w_ref, c0s_ref, c0b_ref, f0, n_img):
    Hs = Ws = 64
    Ms = n_img * Hs * Ws
    y = jnp.dot(col_ref[...], c0w_ref[...], preferred_element_type=jnp.float32)
    y = jnp.maximum(y * c0s_ref[...] + c0b_ref[...], 0.0)
    r = lax.broadcasted_iota(jnp.int32, (Ms, NUM_INIT_FEATURES), 0)
    hh = (r // Ws) % Hs
    ww = r % Ws
    up = pltpu.roll(y, Ws, 0) * (hh >= 1).astype(jnp.float32)
    dn = pltpu.roll(y, Ms - Ws, 0) * (hh < Hs - 1).astype(jnp.float32)
    mv = jnp.maximum(y, jnp.maximum(up, dn))
    lf = pltpu.roll(mv, 1, 0) * (ww >= 1).astype(jnp.float32)
    rt = pltpu.roll(mv, Ms - 1, 0) * (ww < Ws - 1).astype(jnp.float32)
    mh = jnp.maximum(mv, jnp.maximum(lf, rt))
    sel = mh.reshape(n_img * Hs // 2, 2, Ws // 2, 2,
                     NUM_INIT_FEATURES)[:, 0, :, 0, :]
    f0[:, :NUM_INIT_FEATURES] = sel.reshape(Ms // 4, NUM_INIT_FEATURES)


def _dense_layers(feat_ref, sb1_ref, w1_ref, s2_ref, b2_ref, w2_ref,
                  zws_ref, base, L, H, W, n_img):
    M = n_img * H * W
    B = W + 1
    masks = _tap_masks(H, W, M)
    zws_ref[0:B, :] = jnp.zeros((B, 9 * GROWTH), jnp.float32)
    zws_ref[B + M:B + M + B, :] = jnp.zeros((B, 9 * GROWTH), jnp.float32)
    cins = [base + l * GROWTH for l in range(L)]
    offs = np.cumsum([0] + cins[:-1]).tolist()
    for l in range(L):
        cin, off = cins[l], offs[l]
        xin = feat_ref[:, :cin]
        h1 = jnp.maximum(xin * sb1_ref[0:1, off:off + cin]
                         + sb1_ref[1:2, off:off + cin], 0.0)
        y = jnp.dot(h1.astype(jnp.bfloat16), w1_ref[off:off + cin, :],
                    preferred_element_type=jnp.float32)
        z = jnp.maximum(y * s2_ref[l] + b2_ref[l], 0.0)
        wl = jnp.concatenate([w2_ref[l, ti] for ti in range(9)], axis=1)
        zw = jnp.dot(z.astype(jnp.bfloat16), wl,
                     preferred_element_type=jnp.float32)
        zws_ref[B:B + M, :] = zw
        new = zw[:, 4 * GROWTH:5 * GROWTH]
        for ti, (di, dj) in enumerate(TAPS):
            if (di, dj) == (0, 0):
                continue
            sft = di * W + dj
            part = zws_ref[B + sft:B + sft + M,
                           ti * GROWTH:(ti + 1) * GROWTH]
            new = new + part * masks[(di, dj)]
        feat_ref[:, cin:cin + GROWTH] = new


def _avgpool_matmul(feat, ts, tb, tw, n_img, H, W, c_full):
    M = feat.shape[0]
    hf = jnp.maximum(feat * ts + tb, 0.0)
    a = hf + pltpu.roll(hf, M - 1, 0)
    s4 = a + pltpu.roll(a, M - W, 0)
    sel = s4.reshape(n_img * H // 2, 2, W // 2, 2, c_full)[:, 0, :, 0, :]
    pooled = sel.reshape(M // 4, c_full) * 0.25
    return jnp.dot(pooled.astype(jnp.bfloat16), tw,
                   preferred_element_type=jnp.float32)


def _head(feat_ref, n5s_ref, n5b_ref, lw_ref, lb_ref, o_ref, n_img, HW, c_full):
    hf = jnp.maximum(feat_ref[...] * n5s_ref[...] + n5b_ref[...], 0.0)
    pooled = hf.reshape(n_img, HW, c_full).sum(axis=1) / HW
    o_ref[...] = jnp.dot(pooled.astype(jnp.bfloat16), lw_ref[...],
                         preferred_element_type=jnp.float32) + lb_ref[...]


def _full(arr):
    nd = arr.ndim
    return pl.BlockSpec(arr.shape, lambda: (0,) * nd)


def _cp():
    return pltpu.CompilerParams(vmem_limit_bytes=VMEM_LIMIT_BYTES)


def _scr(n_img, s, c):
    return [pltpu.VMEM((n_img * s * s, c), jnp.float32),
            pltpu.VMEM((n_img * s * s + 2 * (s + 1), 9 * GROWTH), jnp.float32)]


def _call_a(col, c0w, c0s, c0b, sb1, w1, s2, b2, w2, ts, tb, tw, n_img):
    base, L, H = NUM_INIT_FEATURES, DB_CONFIG[0], 32
    c_full = base + L * GROWTH

    def body(col_ref, c0w_ref, c0s_ref, c0b_ref, sb1_ref, w1_ref, s2_ref,
             b2_ref, w2_ref, ts_ref, tb_ref, tw_ref, o_ref, f0, zws):
        _stem_compute(col_ref, c0w_ref, c0s_ref, c0b_ref, f0, n_img)
        _dense_layers(f0, sb1_ref, w1_ref, s2_ref, b2_ref, w2_ref, zws,
                      base, L, H, H, n_img)
        o_ref[...] = _avgpool_matmul(f0[...], ts_ref[...], tb_ref[...],
                                     tw_ref[...], n_img, H, H, c_full)

    ins = (col, c0w, c0s, c0b, sb1, w1, s2, b2, w2, ts, tb, tw)
    return pl.pallas_call(
        body,
        out_shape=jax.ShapeDtypeStruct((n_img * (H // 2) ** 2, c_full // 2),
                                       jnp.float32),
        in_specs=[_full(a) for a in ins],
        out_specs=pl.BlockSpec((n_img * (H // 2) ** 2, c_full // 2),
                               lambda: (0, 0)),
        scratch_shapes=_scr(n_img, H, c_full),
        compiler_params=_cp(),
    )(*ins)


def _call_b(feat, sb1_1, w1_1, s2_1, b2_1, w2_1, ts1, tb1, tw1,
            sb1_2, w1_2, s2_2, b2_2, w2_2, ts2, tb2, tw2, n_img):
    base1, L1, H1 = 128, DB_CONFIG[1], 16
    c1 = base1 + L1 * GROWTH
    base2, L2, H2 = c1 // 2, DB_CONFIG[2], 8
    c2 = base2 + L2 * GROWTH

    def body(x_ref, sb1a, w1a, s2a, b2a, w2a, tsa, tba, twa,
             sb1b, w1b, s2b, b2b, w2b, tsb, tbb, twb,
             o_ref, f1, zws1, f2, zws2):
        f1[:, :base1] = x_ref[...]
        _dense_layers(f1, sb1a, w1a, s2a, b2a, w2a, zws1,
                      base1, L1, H1, H1, n_img)
        f2[:, :base2] = _avgpool_matmul(f1[...], tsa[...], tba[...],
                                        twa[...], n_img, H1, H1, c1)
        _dense_layers(f2, sb1b, w1b, s2b, b2b, w2b, zws2,
                      base2, L2, H2, H2, n_img)
        o_ref[...] = _avgpool_matmul(f2[...], tsb[...], tbb[...],
                                     twb[...], n_img, H2, H2, c2)

    ins = (feat, sb1_1, w1_1, s2_1, b2_1, w2_1, ts1, tb1, tw1,
           sb1_2, w1_2, s2_2, b2_2, w2_2, ts2, tb2, tw2)
    return pl.pallas_call(
        body,
        out_shape=jax.ShapeDtypeStruct((n_img * (H2 // 2) ** 2, c2 // 2),
                                       jnp.float32),
        in_specs=[_full(a) for a in ins],
        out_specs=pl.BlockSpec((n_img * (H2 // 2) ** 2, c2 // 2),
                               lambda: (0, 0)),
        scratch_shapes=_scr(n_img, H1, c1) + _scr(n_img, H2, c2),
        compiler_params=_cp(),
    )(*ins)


def _call_c(feat, sb1, w1, s2, b2, w2, n5s, n5b, lw, lb, n_img):
    base, L, H = 640, DB_CONFIG[3], 4
    c_full = base + L * GROWTH

    def body(x_ref, sb1_ref, w1_ref, s2_ref, b2_ref, w2_ref,
             n5s_ref, n5b_ref, lw_ref, lb_ref, o_ref, f3, zws):
        f3[:, :base] = x_ref[...]
        _dense_layers(f3, sb1_ref, w1_ref, s2_ref, b2_ref, w2_ref, zws,
                      base, L, H, H, n_img)
        _head(f3, n5s_ref, n5b_ref, lw_ref, lb_ref, o_ref,
              n_img, H * H, c_full)

    ins = (feat, sb1, w1, s2, b2, w2, n5s, n5b, lw, lb)
    return pl.pallas_call(
        body,
        out_shape=jax.ShapeDtypeStruct((n_img, NUM_CLASSES), jnp.float32),
        in_specs=[_full(a) for a in ins],
        out_specs=pl.BlockSpec((n_img, NUM_CLASSES), lambda: (0, 0)),
        scratch_shapes=_scr(n_img, H, c_full),
        compiler_params=_cp(),
    )(*ins)


def kernel(x, conv0_w, norm0_s, norm0_b,
           sb1_0, w1_0, s2_0, b2_0, w2_0, trans_s_0, trans_b_0, trans_w_0,
           sb1_1, w1_1, s2_1, b2_1, w2_1, trans_s_1, trans_b_1, trans_w_1,
           sb1_2, w1_2, s2_2, b2_2, w2_2, trans_s_2, trans_b_2, trans_w_2,
           sb1_3, w1_3, s2_3, b2_3, w2_3,
           norm5_s, norm5_b, lin_w, lin_b):
    n_img = x.shape[0]
    xh = jnp.transpose(x, (0, 2, 3, 1)).astype(jnp.float32)
    col = _im2col7(xh).astype(jnp.bfloat16)
    feat = _call_a(col, conv0_w, norm0_s, norm0_b,
                   sb1_0, w1_0, s2_0, b2_0, w2_0,
                   trans_s_0, trans_b_0, trans_w_0, n_img)
    feat = _call_b(feat, sb1_1, w1_1, s2_1, b2_1, w2_1,
                   trans_s_1, trans_b_1, trans_w_1,
                   sb1_2, w1_2, s2_2, b2_2, w2_2,
                   trans_s_2, trans_b_2, trans_w_2, n_img)
    return _call_c(feat, sb1_3, w1_3, s2_3, b2_3, w2_3,
                   norm5_s, norm5_b, lin_w, lin_b, n_img)


# speedup vs baseline: 1.8342x; 1.5712x over previous
import numpy as np

import jax
import jax.numpy as jnp
from jax import lax
from jax.experimental import pallas as pl
from jax.experimental.pallas import tpu as pltpu

GROWTH = 32
BOTTLENECK = 128
DB_CONFIG = (6, 12, 32, 32)
NUM_INIT_FEATURES = 64
NUM_CLASSES = 6
VMEM_LIMIT_BYTES = 64 * 1024 * 1024
TAPS = tuple((di, dj) for di in (-1, 0, 1) for dj in (-1, 0, 1))


def _im2col7(x_nhwc):
    N, H, W, C = x_nhwc.shape
    Ho = (H + 6 - 7) // 2 + 1
    p = lax.conv_general_dilated_patches(
        x_nhwc, (7, 7), (2, 2), ((3, 3), (3, 3)),
        dimension_numbers=("NHWC", "HWIO", "NHWC"))
    return p.reshape(N * Ho * Ho, 49 * C)


def _perm_w0(conv0_w):
    return jnp.transpose(conv0_w.reshape(49, 3, NUM_INIT_FEATURES),
                         (1, 0, 2)).reshape(147, NUM_INIT_FEATURES)


def _tap_masks(H, W, M):
    r = lax.broadcasted_iota(jnp.int32, (M, GROWTH), 0)
    hh = (r // W) % H
    ww = r % W
    mh = {-1: (hh >= 1).astype(jnp.float32), 0: None,
          1: (hh < H - 1).astype(jnp.float32)}
    mw = {-1: (ww >= 1).astype(jnp.float32), 0: None,
          1: (ww < W - 1).astype(jnp.float32)}
    masks = {}
    for (di, dj) in TAPS:
        if (di, dj) == (0, 0):
            continue
        a, b = mh[di], mw[dj]
        masks[(di, dj)] = b if a is None else (a if b is None else a * b)
    return masks


def _stem_compute(col_ref, c0w_ref, c0s_ref, c0b_ref, f0, n_img):
    Hs = Ws = 64
    Ms = n_img * Hs * Ws
    y = jnp.dot(col_ref[...], c0w_ref[...], preferred_element_type=jnp.float32)
    y = jnp.maximum(y * c0s_ref[...] + c0b_ref[...], 0.0)
    r = lax.broadcasted_iota(jnp.int32, (Ms, NUM_INIT_FEATURES), 0)
    hh = (r // Ws) % Hs
    ww = r % Ws
    up = pltpu.roll(y, Ws, 0) * (hh >= 1).astype(jnp.float32)
    dn = pltpu.roll(y, Ms - Ws, 0) * (hh < Hs - 1).astype(jnp.float32)
    mv = jnp.maximum(y, jnp.maximum(up, dn))
    lf = pltpu.roll(mv, 1, 0) * (ww >= 1).astype(jnp.float32)
    rt = pltpu.roll(mv, Ms - 1, 0) * (ww < Ws - 1).astype(jnp.float32)
    mh = jnp.maximum(mv, jnp.maximum(lf, rt))
    sel = mh.reshape(n_img * Hs // 2, 2, Ws // 2, 2,
                     NUM_INIT_FEATURES)[:, 0, :, 0, :]
    f0[:, :NUM_INIT_FEATURES] = sel.reshape(Ms // 4, NUM_INIT_FEATURES)


def _dense_layers(feat_ref, sb1_ref, w1_ref, s2_ref, b2_ref, w2_ref,
                  zws_ref, base, L, H, W, n_img):
    M = n_img * H * W
    B = W + 1
    masks = _tap_masks(H, W, M)
    zws_ref[0:B, :] = jnp.zeros((B, 9 * GROWTH), jnp.float32)
    zws_ref[B + M:B + M + B, :] = jnp.zeros((B, 9 * GROWTH), jnp.float32)
    cins = [base + l * GROWTH for l in range(L)]
    offs = np.cumsum([0] + cins[:-1]).tolist()
    for l in range(L):
        cin, off = cins[l], offs[l]
        xin = feat_ref[:, :cin]
        h1 = jnp.maximum(xin * sb1_ref[0:1, off:off + cin]
                         + sb1_ref[1:2, off:off + cin], 0.0)
        y = jnp.dot(h1.astype(jnp.bfloat16), w1_ref[off:off + cin, :],
                    preferred_element_type=jnp.float32)
        z = jnp.maximum(y * s2_ref[l] + b2_ref[l], 0.0)
        wl = jnp.concatenate([w2_ref[l, ti] for ti in range(9)], axis=1)
        zw = jnp.dot(z.astype(jnp.bfloat16), wl,
                     preferred_element_type=jnp.float32)
        zws_ref[B:B + M, :] = zw
        new = zw[:, 4 * GROWTH:5 * GROWTH]
        for ti, (di, dj) in enumerate(TAPS):
            if (di, dj) == (0, 0):
                continue
            sft = di * W + dj
            part = zws_ref[B + sft:B + sft + M,
                           ti * GROWTH:(ti + 1) * GROWTH]
            new = new + part * masks[(di, dj)]
        feat_ref[:, cin:cin + GROWTH] = new


def _avgpool_matmul(feat, ts, tb, tw, n_img, H, W, c_full):
    M = feat.shape[0]
    hf = jnp.maximum(feat * ts + tb, 0.0)
    a = hf + pltpu.roll(hf, M - 1, 0)
    s4 = a + pltpu.roll(a, M - W, 0)
    sel = s4.reshape(n_img * H // 2, 2, W // 2, 2, c_full)[:, 0, :, 0, :]
    pooled = sel.reshape(M // 4, c_full) * 0.25
    return jnp.dot(pooled.astype(jnp.bfloat16), tw,
                   preferred_element_type=jnp.float32)


def _head(feat_ref, n5s_ref, n5b_ref, lw_ref, lb_ref, o_ref, n_img, HW, c_full):
    hf = jnp.maximum(feat_ref[...] * n5s_ref[...] + n5b_ref[...], 0.0)
    pooled = hf.reshape(n_img, HW, c_full).sum(axis=1) / HW
    o_ref[...] = jnp.dot(pooled.astype(jnp.bfloat16), lw_ref[...],
                         preferred_element_type=jnp.float32) + lb_ref[...]


def _full(arr):
    nd = arr.ndim
    return pl.BlockSpec(arr.shape, lambda: (0,) * nd)


def _cp():
    return pltpu.CompilerParams(vmem_limit_bytes=VMEM_LIMIT_BYTES)


def _scr(n_img, s, c):
    return [pltpu.VMEM((n_img * s * s, c), jnp.float32),
            pltpu.VMEM((n_img * s * s + 2 * (s + 1), 9 * GROWTH), jnp.float32)]


def _call_a(col, c0w, c0s, c0b, sb1, w1, s2, b2, w2, ts, tb, tw, n_img):
    base, L, H = NUM_INIT_FEATURES, DB_CONFIG[0], 32
    c_full = base + L * GROWTH

    def body(col_ref, c0w_ref, c0s_ref, c0b_ref, sb1_ref, w1_ref, s2_ref,
             b2_ref, w2_ref, ts_ref, tb_ref, tw_ref, o_ref, f0, zws):
        _stem_compute(col_ref, c0w_ref, c0s_ref, c0b_ref, f0, n_img)
        _dense_layers(f0, sb1_ref, w1_ref, s2_ref, b2_ref, w2_ref, zws,
                      base, L, H, H, n_img)
        o_ref[...] = _avgpool_matmul(f0[...], ts_ref[...], tb_ref[...],
                                     tw_ref[...], n_img, H, H, c_full)

    ins = (col, c0w, c0s, c0b, sb1, w1, s2, b2, w2, ts, tb, tw)
    return pl.pallas_call(
        body,
        out_shape=jax.ShapeDtypeStruct((n_img * (H // 2) ** 2, c_full // 2),
                                       jnp.float32),
        in_specs=[_full(a) for a in ins],
        out_specs=pl.BlockSpec((n_img * (H // 2) ** 2, c_full // 2),
                               lambda: (0, 0)),
        scratch_shapes=_scr(n_img, H, c_full),
        compiler_params=_cp(),
    )(*ins)


def _call_b(feat, sb1_1, w1_1, s2_1, b2_1, w2_1, ts1, tb1, tw1,
            sb1_2, w1_2, s2_2, b2_2, w2_2, ts2, tb2, tw2, n_img):
    base1, L1, H1 = 128, DB_CONFIG[1], 16
    c1 = base1 + L1 * GROWTH
    base2, L2, H2 = c1 // 2, DB_CONFIG[2], 8
    c2 = base2 + L2 * GROWTH

    def body(x_ref, sb1a, w1a, s2a, b2a, w2a, tsa, tba, twa,
             sb1b, w1b, s2b, b2b, w2b, tsb, tbb, twb,
             o_ref, f1, zws1, f2, zws2):
        f1[:, :base1] = x_ref[...]
        _dense_layers(f1, sb1a, w1a, s2a, b2a, w2a, zws1,
                      base1, L1, H1, H1, n_img)
        f2[:, :base2] = _avgpool_matmul(f1[...], tsa[...], tba[...],
                                        twa[...], n_img, H1, H1, c1)
        _dense_layers(f2, sb1b, w1b, s2b, b2b, w2b, zws2,
                      base2, L2, H2, H2, n_img)
        o_ref[...] = _avgpool_matmul(f2[...], tsb[...], tbb[...],
                                     twb[...], n_img, H2, H2, c2)

    ins = (feat, sb1_1, w1_1, s2_1, b2_1, w2_1, ts1, tb1, tw1,
           sb1_2, w1_2, s2_2, b2_2, w2_2, ts2, tb2, tw2)
    return pl.pallas_call(
        body,
        out_shape=jax.ShapeDtypeStruct((n_img * (H2 // 2) ** 2, c2 // 2),
                                       jnp.float32),
        in_specs=[_full(a) for a in ins],
        out_specs=pl.BlockSpec((n_img * (H2 // 2) ** 2, c2 // 2),
                               lambda: (0, 0)),
        scratch_shapes=_scr(n_img, H1, c1) + _scr(n_img, H2, c2),
        compiler_params=_cp(),
    )(*ins)


def _call_c(feat, sb1, w1, s2, b2, w2, n5s, n5b, lw, lb, n_img):
    base, L, H = 640, DB_CONFIG[3], 4
    c_full = base + L * GROWTH

    def body(x_ref, sb1_ref, w1_ref, s2_ref, b2_ref, w2_ref,
             n5s_ref, n5b_ref, lw_ref, lb_ref, o_ref, f3, zws):
        f3[:, :base] = x_ref[...]
        _dense_layers(f3, sb1_ref, w1_ref, s2_ref, b2_ref, w2_ref, zws,
                      base, L, H, H, n_img)
        _head(f3, n5s_ref, n5b_ref, lw_ref, lb_ref, o_ref,
              n_img, H * H, c_full)

    ins = (feat, sb1, w1, s2, b2, w2, n5s, n5b, lw, lb)
    return pl.pallas_call(
        body,
        out_shape=jax.ShapeDtypeStruct((n_img, NUM_CLASSES), jnp.float32),
        in_specs=[_full(a) for a in ins],
        out_specs=pl.BlockSpec((n_img, NUM_CLASSES), lambda: (0, 0)),
        scratch_shapes=_scr(n_img, H, c_full),
        compiler_params=_cp(),
    )(*ins)


def kernel(x, conv0_w, norm0_s, norm0_b,
           sb1_0, w1_0, s2_0, b2_0, w2_0, trans_s_0, trans_b_0, trans_w_0,
           sb1_1, w1_1, s2_1, b2_1, w2_1, trans_s_1, trans_b_1, trans_w_1,
           sb1_2, w1_2, s2_2, b2_2, w2_2, trans_s_2, trans_b_2, trans_w_2,
           sb1_3, w1_3, s2_3, b2_3, w2_3,
           norm5_s, norm5_b, lin_w, lin_b):
    n_img = x.shape[0]
    xh = jnp.transpose(x, (0, 2, 3, 1)).astype(jnp.float32)
    col = _im2col7(xh).astype(jnp.bfloat16)
    feat = _call_a(col, _perm_w0(conv0_w), norm0_s, norm0_b,
                   sb1_0, w1_0, s2_0, b2_0, w2_0,
                   trans_s_0, trans_b_0, trans_w_0, n_img)
    feat = _call_b(feat, sb1_1, w1_1, s2_1, b2_1, w2_1,
                   trans_s_1, trans_b_1, trans_w_1,
                   sb1_2, w1_2, s2_2, b2_2, w2_2,
                   trans_s_2, trans_b_2, trans_w_2, n_img)
    return _call_c(feat, sb1_3, w1_3, s2_3, b2_3, w2_3,
                   norm5_s, norm5_b, lin_w, lin_b, n_img)
